# baseline (speedup 1.0000x reference)
import jax
import jax.numpy as jnp
from jax.experimental import pallas as pl
from jax.experimental.pallas import tpu as pltpu

K = 1024
D = 64
SS = 512
NSS = K // SS
RG = 8
NRG = SS // RG


def _vq_body(x_ref, w_ref, zsq_ref, esq_ref, out_ref):
    x = x_ref[...]
    z_sq = zsq_ref[...]
    BN = x.shape[0]

    m = None
    c = None
    for ss in range(NSS):
        ws = w_ref[ss * SS:(ss + 1) * SS, :]
        w2s = ws + ws
        p = jax.lax.dot_general(w2s, x, (((1,), (1,)), ((), ())),
                                preferred_element_type=jnp.float32)
        for g in range(NRG):
            pg = p[g * RG:(g + 1) * RG, :]
            eg = esq_ref[ss * SS + g * RG:ss * SS + (g + 1) * RG, :]
            dg = (z_sq - pg) + eg
            if m is None:
                m = dg
                c = jnp.zeros((RG, BN), jnp.int32)
            else:
                lt = dg < m
                m = jnp.where(lt, dg, m)
                c = jnp.where(lt, ss * NRG + g, c)

    r = jax.lax.broadcasted_iota(jnp.int32, (RG, BN), 0)
    kfull = c * RG + r
    m_min = jnp.min(m, axis=0, keepdims=True)
    kcand = jnp.where(m == m_min, kfull, K)
    out_ref[...] = jnp.min(kcand, axis=0, keepdims=True)


def kernel(z_e, W):
    B, S, d = z_e.shape
    N = B * S
    flat = z_e.reshape(N, d)
    z_sq = jnp.sum(flat * flat, axis=1, keepdims=True)
    e_sq = jnp.sum(W * W, axis=1)
    BN = 16 * S
    grid = (N // BN,)
    out = pl.pallas_call(
        _vq_body,
        grid=grid,
        in_specs=[
            pl.BlockSpec((BN, d), lambda i: (i, 0)),
            pl.BlockSpec((K, d), lambda i: (0, 0)),
            pl.BlockSpec((1, BN), lambda i: (0, i)),
            pl.BlockSpec((K, 1), lambda i: (0, 0)),
        ],
        out_specs=pl.BlockSpec((1, BN), lambda i: (0, i)),
        out_shape=jax.ShapeDtypeStruct((1, N), jnp.int32),
        compiler_params=pltpu.CompilerParams(
            dimension_semantics=("parallel",)),
    )(flat, W, z_sq.reshape(1, N), e_sq.reshape(K, 1))
    return out.reshape(B, S)


# speedup vs baseline: 1.0482x; 1.0482x over previous
import jax
import jax.numpy as jnp
from jax.experimental import pallas as pl
from jax.experimental.pallas import tpu as pltpu

K = 1024
D = 64
SS = 512
NSS = K // SS
RG = 8
NRG = SS // RG


def _vq_body(x_ref, w_ref, zsq_ref, esq_ref, out_ref):
    x = x_ref[...]
    z_sq = zsq_ref[...]
    BN = x.shape[0]

    m = None
    c = None
    for ss in range(NSS):
        ws = w_ref[ss * SS:(ss + 1) * SS, :]
        w2s = ws + ws
        p = jax.lax.dot_general(w2s, x, (((1,), (1,)), ((), ())),
                                preferred_element_type=jnp.float32)
        for g in range(NRG):
            pg = p[g * RG:(g + 1) * RG, :]
            eg = esq_ref[ss * SS + g * RG:ss * SS + (g + 1) * RG, :]
            dg = (z_sq - pg) + eg
            if m is None:
                m = dg
                c = jnp.zeros((RG, BN), jnp.int32)
            else:
                lt = dg < m
                m = jnp.where(lt, dg, m)
                c = jnp.where(lt, ss * NRG + g, c)

    r = jax.lax.broadcasted_iota(jnp.int32, (RG, BN), 0)
    kfull = c * RG + r
    m_min = jnp.min(m, axis=0, keepdims=True)
    kcand = jnp.where(m == m_min, kfull, K)
    out_ref[...] = jnp.min(kcand, axis=0, keepdims=True)


def kernel(z_e, W):
    B, S, d = z_e.shape
    N = B * S
    flat = z_e.reshape(N, d)
    z_sq = jnp.sum(flat * flat, axis=1, keepdims=True)
    e_sq = jnp.sum(W * W, axis=1)
    BN = 4 * S
    grid = (N // BN,)
    out = pl.pallas_call(
        _vq_body,
        grid=grid,
        in_specs=[
            pl.BlockSpec((BN, d), lambda i: (i, 0)),
            pl.BlockSpec((K, d), lambda i: (0, 0)),
            pl.BlockSpec((1, BN), lambda i: (0, i)),
            pl.BlockSpec((K, 1), lambda i: (0, 0)),
        ],
        out_specs=pl.BlockSpec((1, BN), lambda i: (0, i)),
        out_shape=jax.ShapeDtypeStruct((1, N), jnp.int32),
        compiler_params=pltpu.CompilerParams(
            dimension_semantics=("parallel",)),
    )(flat, W, z_sq.reshape(1, N), e_sq.reshape(K, 1))
    return out.reshape(B, S)


# speedup vs baseline: 1.0513x; 1.0030x over previous
import jax
import jax.numpy as jnp
from jax.experimental import pallas as pl
from jax.experimental.pallas import tpu as pltpu

K = 1024
D = 64
SS = 512
NSS = K // SS
RG = 8
NRG = SS // RG


def _vq_body(x_ref, w_ref, zsq_ref, esq_ref, out_ref):
    x = x_ref[...]
    z_sq = zsq_ref[...]
    BN = x.shape[0]

    m = None
    c = None
    for ss in range(NSS):
        ws = w_ref[ss * SS:(ss + 1) * SS, :]
        w2s = ws + ws
        p = jax.lax.dot_general(w2s, x, (((1,), (1,)), ((), ())),
                                preferred_element_type=jnp.float32)
        for g in range(NRG):
            pg = p[g * RG:(g + 1) * RG, :]
            eg = esq_ref[ss * SS + g * RG:ss * SS + (g + 1) * RG, :]
            dg = (z_sq - pg) + eg
            if m is None:
                m = dg
                c = jnp.zeros((RG, BN), jnp.int32)
            else:
                lt = dg < m
                m = jnp.minimum(dg, m)
                c = jnp.where(lt, ss * NRG + g, c)

    r = jax.lax.broadcasted_iota(jnp.int32, (RG, BN), 0)
    kfull = c * RG + r
    m_min = jnp.min(m, axis=0, keepdims=True)
    kcand = jnp.where(m == m_min, kfull, K)
    out_ref[...] = jnp.min(kcand, axis=0, keepdims=True)


def kernel(z_e, W):
    B, S, d = z_e.shape
    N = B * S
    flat = z_e.reshape(N, d)
    z_sq = jnp.sum(flat * flat, axis=1, keepdims=True)
    e_sq = jnp.sum(W * W, axis=1)
    BN = 4 * S
    grid = (N // BN,)
    out = pl.pallas_call(
        _vq_body,
        grid=grid,
        in_specs=[
            pl.BlockSpec((BN, d), lambda i: (i, 0)),
            pl.BlockSpec((K, d), lambda i: (0, 0)),
            pl.BlockSpec((1, BN), lambda i: (0, i)),
            pl.BlockSpec((K, 1), lambda i: (0, 0)),
        ],
        out_specs=pl.BlockSpec((1, BN), lambda i: (0, i)),
        out_shape=jax.ShapeDtypeStruct((1, N), jnp.int32),
        compiler_params=pltpu.CompilerParams(
            dimension_semantics=("parallel",)),
    )(flat, W, z_sq.reshape(1, N), e_sq.reshape(K, 1))
    return out.reshape(B, S)
